# Initial kernel scaffold; baseline (speedup 1.0000x reference)
import jax
import jax.numpy as jnp
from jax.experimental import pallas as pl


def kernel(x, W_self, b_self, W_nb, b_nb, W_comb, b_comb, gamma, beta):
    raise NotImplementedError("write your pallas kernel here")


# speedup vs baseline: 17.5363x; 17.5363x over previous
import jax
import jax.numpy as jnp
from jax.experimental import pallas as pl

_K = 9
_EPS = 1e-5


def _fuse_kernel(ws_ref, bs_ref, wn_ref, bn_ref, wc_ref, bc_ref,
                 fs_ref, fn_ref, c0_ref):
    c = ws_ref.shape[0]
    wc1 = wc_ref[:, :c]
    wc2 = wc_ref[:, c:]
    fs_ref[...] = jnp.dot(wc1, ws_ref[...], preferred_element_type=jnp.float32)
    fn_ref[...] = jnp.dot(wc2, wn_ref[...], preferred_element_type=jnp.float32)
    c0_ref[...] = (bc_ref[...]
                   + jnp.dot(wc1, bs_ref[...], preferred_element_type=jnp.float32)
                   + jnp.dot(wc2, bn_ref[...], preferred_element_type=jnp.float32))


def _main_kernel(xb_ref, fs_ref, fn_ref, c0_ref, pre_ref):
    xb = xb_ref[0]
    n = xb.shape[1]
    g = jax.lax.dot_general(xb, xb, (((0,), (0,)), ((), ())),
                            preferred_element_type=jnp.float32)
    sq = jnp.sum(xb * xb, axis=0, keepdims=True)
    d = sq - 2.0 * g
    m = jnp.zeros((n, n), jnp.float32)
    iota = jax.lax.broadcasted_iota(jnp.int32, (n, n), 1)
    inf = jnp.float32(jnp.inf)
    for _ in range(_K):
        mv = jnp.min(d, axis=1, keepdims=True)
        eq = d == mv
        mi = jnp.min(jnp.where(eq, iota, n), axis=1, keepdims=True)
        sel = iota == mi
        m = jnp.where(sel, 1.0, m)
        d = jnp.where(sel, inf, d)
    nbt = jax.lax.dot_general(xb, m, (((1,), (1,)), ((), ())),
                              preferred_element_type=jnp.float32)
    out_t = (jnp.dot(fs_ref[...], xb, preferred_element_type=jnp.float32)
             + jnp.dot(fn_ref[...], nbt * (1.0 / _K),
                       preferred_element_type=jnp.float32)
             + c0_ref[...])
    pre_ref[0] = out_t


def _bn_kernel(pre_ref, x_ref, gamma_ref, beta_ref, out_ref):
    pre = pre_ref[...]
    b, ct, n = pre.shape
    cnt = b * n
    s1 = jnp.sum(pre, axis=(0, 2), keepdims=True) / cnt
    s2 = jnp.sum(pre * pre, axis=(0, 2), keepdims=True) / cnt
    var = s2 - s1 * s1
    inv = jax.lax.rsqrt(var + _EPS)
    scale = gamma_ref[...][None, :, :] * inv
    shift = beta_ref[...][None, :, :] - s1 * scale
    out_ref[...] = jnp.maximum(pre * scale + shift + x_ref[...], 0.0)


def kernel(x, W_self, b_self, W_nb, b_nb, W_comb, b_comb, gamma, beta):
    B, C, H, W = x.shape
    N = H * W
    xr = x.reshape(B, C, N)
    bs2 = b_self[:, None]
    bn2 = b_nb[:, None]
    bc2 = b_comb[:, None]
    f_self, f_nb, c0 = pl.pallas_call(
        _fuse_kernel,
        out_shape=(
            jax.ShapeDtypeStruct((C, C), jnp.float32),
            jax.ShapeDtypeStruct((C, C), jnp.float32),
            jax.ShapeDtypeStruct((C, 1), jnp.float32),
        ),
    )(W_self, bs2, W_nb, bn2, W_comb, bc2)

    pre = pl.pallas_call(
        _main_kernel,
        grid=(B,),
        in_specs=[
            pl.BlockSpec((1, C, N), lambda b: (b, 0, 0)),
            pl.BlockSpec((C, C), lambda b: (0, 0)),
            pl.BlockSpec((C, C), lambda b: (0, 0)),
            pl.BlockSpec((C, 1), lambda b: (0, 0)),
        ],
        out_specs=pl.BlockSpec((1, C, N), lambda b: (b, 0, 0)),
        out_shape=jax.ShapeDtypeStruct((B, C, N), jnp.float32),
    )(xr, f_self, f_nb, c0)

    CT = 128
    out = pl.pallas_call(
        _bn_kernel,
        grid=(C // CT,),
        in_specs=[
            pl.BlockSpec((B, CT, N), lambda c: (0, c, 0)),
            pl.BlockSpec((B, CT, N), lambda c: (0, c, 0)),
            pl.BlockSpec((CT, 1), lambda c: (c, 0)),
            pl.BlockSpec((CT, 1), lambda c: (c, 0)),
        ],
        out_specs=pl.BlockSpec((B, CT, N), lambda c: (0, c, 0)),
        out_shape=jax.ShapeDtypeStruct((B, C, N), jnp.float32),
    )(pre, xr, gamma[:, None], beta[:, None])
    return out.reshape(B, C, H, W)


# speedup vs baseline: 28.7189x; 1.6377x over previous
import jax
import jax.numpy as jnp
from jax.experimental import pallas as pl

_K = 9
_EPS = 1e-5


def _fuse_kernel(ws_ref, bs_ref, wn_ref, bn_ref, wc_ref, bc_ref,
                 fs_ref, fn_ref, c0_ref):
    c = ws_ref.shape[0]
    wc1 = wc_ref[:, :c]
    wc2 = wc_ref[:, c:]
    fs_ref[...] = jnp.dot(wc1, ws_ref[...], preferred_element_type=jnp.float32)
    fn_ref[...] = jnp.dot(wc2, wn_ref[...], preferred_element_type=jnp.float32)
    c0_ref[...] = (bc_ref[...]
                   + jnp.dot(wc1, bs_ref[...], preferred_element_type=jnp.float32)
                   + jnp.dot(wc2, bn_ref[...], preferred_element_type=jnp.float32))


def _main_kernel(xb_ref, fs_ref, fn_ref, c0_ref, pre_ref):
    xb = xb_ref[0]
    n = xb.shape[1]
    g = jax.lax.dot_general(xb, xb, (((0,), (0,)), ((), ())),
                            preferred_element_type=jnp.float32)
    sq = jnp.sum(xb * xb, axis=0, keepdims=True)
    d = sq - 2.0 * g
    inf = jnp.float32(jnp.inf)
    rows = jax.lax.broadcasted_iota(jnp.int32, (n, n), 0)
    cols = jax.lax.broadcasted_iota(jnp.int32, (n, n), 1)
    d = jnp.where(rows == cols, inf, d)
    for _ in range(_K - 1):
        mv = jnp.min(d, axis=1, keepdims=True)
        d = jnp.where(d == mv, inf, d)
    m = (d == inf).astype(jnp.bfloat16)
    xb16 = xb.astype(jnp.bfloat16)
    nbt = jax.lax.dot_general(xb16, m, (((1,), (1,)), ((), ())),
                              preferred_element_type=jnp.float32)
    out_t = (jnp.dot(fs_ref[...].astype(jnp.bfloat16), xb16,
                     preferred_element_type=jnp.float32)
             + jnp.dot(fn_ref[...].astype(jnp.bfloat16),
                       (nbt * (1.0 / _K)).astype(jnp.bfloat16),
                       preferred_element_type=jnp.float32)
             + c0_ref[...])
    pre_ref[0] = out_t


def _bn_kernel(pre_ref, x_ref, gamma_ref, beta_ref, out_ref):
    pre = pre_ref[...]
    b, ct, n = pre.shape
    cnt = b * n
    s1 = jnp.sum(pre, axis=(0, 2), keepdims=True) / cnt
    s2 = jnp.sum(pre * pre, axis=(0, 2), keepdims=True) / cnt
    var = s2 - s1 * s1
    inv = jax.lax.rsqrt(var + _EPS)
    scale = gamma_ref[...][None, :, :] * inv
    shift = beta_ref[...][None, :, :] - s1 * scale
    out_ref[...] = jnp.maximum(pre * scale + shift + x_ref[...], 0.0)


def kernel(x, W_self, b_self, W_nb, b_nb, W_comb, b_comb, gamma, beta):
    B, C, H, W = x.shape
    N = H * W
    xr = x.reshape(B, C, N)
    bs2 = b_self[:, None]
    bn2 = b_nb[:, None]
    bc2 = b_comb[:, None]
    f_self, f_nb, c0 = pl.pallas_call(
        _fuse_kernel,
        out_shape=(
            jax.ShapeDtypeStruct((C, C), jnp.float32),
            jax.ShapeDtypeStruct((C, C), jnp.float32),
            jax.ShapeDtypeStruct((C, 1), jnp.float32),
        ),
    )(W_self, bs2, W_nb, bn2, W_comb, bc2)

    pre = pl.pallas_call(
        _main_kernel,
        grid=(B,),
        in_specs=[
            pl.BlockSpec((1, C, N), lambda b: (b, 0, 0)),
            pl.BlockSpec((C, C), lambda b: (0, 0)),
            pl.BlockSpec((C, C), lambda b: (0, 0)),
            pl.BlockSpec((C, 1), lambda b: (0, 0)),
        ],
        out_specs=pl.BlockSpec((1, C, N), lambda b: (b, 0, 0)),
        out_shape=jax.ShapeDtypeStruct((B, C, N), jnp.float32),
    )(xr, f_self, f_nb, c0)

    CT = 128
    out = pl.pallas_call(
        _bn_kernel,
        grid=(C // CT,),
        in_specs=[
            pl.BlockSpec((B, CT, N), lambda c: (0, c, 0)),
            pl.BlockSpec((B, CT, N), lambda c: (0, c, 0)),
            pl.BlockSpec((CT, 1), lambda c: (c, 0)),
            pl.BlockSpec((CT, 1), lambda c: (c, 0)),
        ],
        out_specs=pl.BlockSpec((B, CT, N), lambda c: (0, c, 0)),
        out_shape=jax.ShapeDtypeStruct((B, C, N), jnp.float32),
    )(pre, xr, gamma[:, None], beta[:, None])
    return out.reshape(B, C, H, W)


# speedup vs baseline: 28.7246x; 1.0002x over previous
import jax
import jax.numpy as jnp
from jax.experimental import pallas as pl
from jax.experimental.pallas import tpu as pltpu

_K = 9
_EPS = 1e-5


def _fuse_kernel(ws_ref, bs_ref, wn_ref, bn_ref, wc_ref, bc_ref,
                 fs_ref, fn_ref, c0_ref):
    c = ws_ref.shape[0]
    wc1 = wc_ref[:, :c]
    wc2 = wc_ref[:, c:]
    fs_ref[...] = jnp.dot(wc1, ws_ref[...], preferred_element_type=jnp.float32)
    fn_ref[...] = jnp.dot(wc2, wn_ref[...], preferred_element_type=jnp.float32)
    c0_ref[...] = (bc_ref[...]
                   + jnp.dot(wc1, bs_ref[...], preferred_element_type=jnp.float32)
                   + jnp.dot(wc2, bn_ref[...], preferred_element_type=jnp.float32))


def _gram(xv):
    return jax.lax.dot_general(xv * -2.0, xv, (((0,), (0,)), ((), ())),
                               preferred_element_type=jnp.float32)


def _main_kernel(xb_ref, xn_ref, fs_ref, fn_ref, c0_ref, gamma_ref, beta_ref,
                 out_ref, g_scr, sq_scr, pre_scr, stats_scr, ss_scr):
    p = pl.program_id(0)
    b = pl.program_id(1)
    n_b = pl.num_programs(1)
    c, n = xb_ref.shape[1], xb_ref.shape[2]
    inf = jnp.float32(jnp.inf)

    @pl.when(p == 0)
    def _phase0():
        cur = jax.lax.rem(b, 2)
        nxt = jax.lax.rem(b + 1, 2)

        @pl.when(b == 0)
        def _prologue():
            xv = xb_ref[0]
            g_scr[0] = _gram(xv)
            sq_scr[0] = jnp.sum(xv * xv, axis=0, keepdims=True)

        xv_n = xn_ref[0]
        g_scr[nxt] = _gram(xv_n)
        sq_scr[nxt] = jnp.sum(xv_n * xv_n, axis=0, keepdims=True)

        rows = jax.lax.broadcasted_iota(jnp.int32, (n, n), 0)
        cols = jax.lax.broadcasted_iota(jnp.int32, (n, n), 1)
        d = jnp.where(rows == cols, inf, sq_scr[cur] + g_scr[cur])
        for _ in range(_K - 1):
            mv = jnp.min(d, axis=1, keepdims=True)
            d = jnp.where(d == mv, inf, d)
        m = (d == inf).astype(jnp.bfloat16)

        xb16 = xb_ref[0].astype(jnp.bfloat16)
        nbt = jax.lax.dot_general(xb16, m, (((1,), (1,)), ((), ())),
                                  preferred_element_type=jnp.float32)
        out_t = (jnp.dot(fs_ref[...].astype(jnp.bfloat16), xb16,
                         preferred_element_type=jnp.float32)
                 + jnp.dot(fn_ref[...].astype(jnp.bfloat16),
                           (nbt * (1.0 / _K)).astype(jnp.bfloat16),
                           preferred_element_type=jnp.float32)
                 + c0_ref[...])
        pre_scr[b] = out_t
        part = jnp.concatenate(
            [jnp.sum(out_t, axis=1, keepdims=True),
             jnp.sum(out_t * out_t, axis=1, keepdims=True)], axis=1)
        stats_scr[...] = jnp.where(b == 0, part, stats_scr[...] + part)

    @pl.when((p == 1) & (b == 0))
    def _finalize_stats():
        cnt = jnp.float32(n_b * n)
        mean = stats_scr[:, 0:1] / cnt
        var = stats_scr[:, 1:2] / cnt - mean * mean
        inv = jax.lax.rsqrt(var + _EPS)
        scale = gamma_ref[...] * inv
        ss_scr[...] = jnp.concatenate(
            [scale, beta_ref[...] - mean * scale], axis=1)

    @pl.when(p == 1)
    def _phase1():
        scale = ss_scr[:, 0:1]
        shift = ss_scr[:, 1:2]
        out_ref[0] = jnp.maximum(pre_scr[b] * scale + shift + xb_ref[0], 0.0)


def kernel(x, W_self, b_self, W_nb, b_nb, W_comb, b_comb, gamma, beta):
    B, C, H, W = x.shape
    N = H * W
    xr = x.reshape(B, C, N)
    f_self, f_nb, c0 = pl.pallas_call(
        _fuse_kernel,
        out_shape=(
            jax.ShapeDtypeStruct((C, C), jnp.float32),
            jax.ShapeDtypeStruct((C, C), jnp.float32),
            jax.ShapeDtypeStruct((C, 1), jnp.float32),
        ),
    )(W_self, b_self[:, None], W_nb, b_nb[:, None], W_comb, b_comb[:, None])

    out = pl.pallas_call(
        _main_kernel,
        grid=(2, B),
        in_specs=[
            pl.BlockSpec((1, C, N), lambda p, b: (b, 0, 0)),
            pl.BlockSpec((1, C, N),
                         lambda p, b: (jnp.where(
                             p == 0, jnp.minimum(b + 1, B - 1), B - 1), 0, 0)),
            pl.BlockSpec((C, C), lambda p, b: (0, 0)),
            pl.BlockSpec((C, C), lambda p, b: (0, 0)),
            pl.BlockSpec((C, 1), lambda p, b: (0, 0)),
            pl.BlockSpec((C, 1), lambda p, b: (0, 0)),
            pl.BlockSpec((C, 1), lambda p, b: (0, 0)),
        ],
        out_specs=pl.BlockSpec((1, C, N),
                               lambda p, b: (jnp.where(p == 0, 0, b), 0, 0)),
        out_shape=jax.ShapeDtypeStruct((B, C, N), jnp.float32),
        scratch_shapes=[
            pltpu.VMEM((2, N, N), jnp.float32),
            pltpu.VMEM((2, 1, N), jnp.float32),
            pltpu.VMEM((B, C, N), jnp.float32),
            pltpu.VMEM((C, 2), jnp.float32),
            pltpu.VMEM((C, 2), jnp.float32),
        ],
    )(xr, xr, f_self, f_nb, c0, gamma[:, None], beta[:, None])
    return out.reshape(B, C, H, W)
